# baseline (speedup 1.0000x reference)
import jax
import jax.numpy as jnp
from jax import lax
from jax.experimental import pallas as pl
from jax.experimental.pallas import tpu as pltpu
from jax.experimental.pallas import tpu_sc as plsc

H = 2048
N = 32768
S = 18432
NSC = N - S
BLK = 512
NC = 2
NS = 16
NW = NC * NS
RPW = NSC // NW
L = 16
C = 8
NB = 4
ZC = 8
NZCH = RPW // C
ZCH = RPW // ZC
HV = H // L
UN = 8


def _tc_body(x_ref, m_ref, o_ref):
    o_ref[...] = jnp.where(
        m_ref[...] != 0, jnp.maximum(x_ref[...], 0.0), 0.0)


def _sc_body(x_hbm, m_hbm, out_hbm, mask_v, nz_idx, nz_idx_a, z_idx, buf, zbuf,
             gs0, gs1, gs2, gs3, ss0, ss1, ss2, ss3, zsem):
    gsems = (gs0, gs1, gs2, gs3)
    ssems = (ss0, ss1, ss2, ss3)
    wid = lax.axis_index("s") * NC + lax.axis_index("c")
    base = S + wid * RPW

    pltpu.sync_copy(m_hbm.at[pl.ds(base, RPW)], mask_v)

    def compact(i, carry):
        nz_off, z_off, any_nz, any_z = carry
        mv = mask_v[pl.ds(i * L, L)]
        rows = (base - S) + i * L + lax.iota(jnp.int32, L)
        pred = mv != 0
        predi = pred.astype(jnp.int32)
        cum = plsc.cumsum(predi)
        cnt = cum[L - 1]
        gpos = nz_off + cum - 1
        plsc.store_scatter(
            nz_idx,
            [jnp.right_shift(gpos, 3), jnp.bitwise_and(gpos, 7)],
            rows, mask=pred)
        plsc.store_scatter(
            nz_idx_a,
            [jnp.right_shift(gpos, 3), jnp.bitwise_and(gpos, 7)],
            rows + S, mask=pred)
        zpred = jnp.logical_not(pred)
        zpos = z_off + plsc.cumsum(1 - predi) - 1
        plsc.store_scatter(
            z_idx,
            [jnp.right_shift(zpos, 3), jnp.bitwise_and(zpos, 7)],
            rows, mask=zpred)
        any_nz = jnp.maximum(
            any_nz, plsc.cummax(jnp.where(pred, rows, -1))[L - 1])
        any_z = jnp.maximum(
            any_z, plsc.cummax(jnp.where(zpred, rows, -1))[L - 1])
        return nz_off + cnt, z_off + (L - cnt), any_nz, any_z

    nz, z, any_nz, any_z = lax.fori_loop(
        0, RPW // L, compact,
        (jnp.int32(0), jnp.int32(0), jnp.int32(-1), jnp.int32(-1)))

    lanes = lax.iota(jnp.int32, L)

    @pl.when((nz & (C - 1)) != 0)
    def _():
        fill = jnp.zeros((L,), jnp.int32) + any_nz
        row = jnp.zeros((L,), jnp.int32) + jnp.right_shift(nz, 3)
        padmask = jnp.logical_and(lanes >= (nz & (C - 1)), lanes < C)
        plsc.store_scatter(nz_idx, [row, lanes], fill, mask=padmask)
        plsc.store_scatter(nz_idx_a, [row, lanes], fill + S, mask=padmask)

    @pl.when((z & (ZC - 1)) != 0)
    def _():
        fill = jnp.zeros((L,), jnp.int32) + any_z
        row = jnp.zeros((L,), jnp.int32) + jnp.right_shift(z, 3)
        plsc.store_scatter(
            z_idx, [row, lanes], fill,
            mask=jnp.logical_and(lanes >= (z & (ZC - 1)), lanes < ZC))

    nzch = jnp.right_shift(nz + (C - 1), 3)
    zch = jnp.right_shift(z + (ZC - 1), 3)

    zero = jnp.zeros((L,), jnp.float32)
    for r in range(ZC):
        def zinit(h, _, r=r):
            zbuf[r, pl.ds(h * L, L)] = zero
            return 0
        lax.fori_loop(0, HV, zinit, 0)

    @pl.when(nzch > 0)
    def _():
        pltpu.async_copy(x_hbm.at[nz_idx_a.at[0]], buf.at[0], gsems[0])

    def group(g, _):
        for s in range(NB):
            j = g * NB + s

            @pl.when(j < nzch)
            def _(j=j, s=s):
                s1 = (s + 1) % NB

                @pl.when(j + 1 < nzch)
                def _():
                    @pl.when(j + 1 >= NB)
                    def _():
                        pltpu.make_async_copy(
                            buf.at[s1], out_hbm.at[nz_idx.at[j + 1 - NB]],
                            ssems[s1]).wait()
                    pltpu.async_copy(
                        x_hbm.at[nz_idx_a.at[j + 1]], buf.at[s1], gsems[s1])

                for t in range(2):
                    @pl.when(2 * j + t < zch)
                    def _(t=t):
                        pltpu.async_copy(
                            zbuf, out_hbm.at[z_idx.at[2 * j + t]], zsem)

                pltpu.make_async_copy(
                    x_hbm.at[nz_idx_a.at[j]], buf.at[s], gsems[s]).wait()
                for r in range(C):
                    def relu(h, _, r=r, s=s):
                        for u in range(UN):
                            off = h * (L * UN) + u * L
                            v = buf[s, r, pl.ds(off, L)]
                            buf[s, r, pl.ds(off, L)] = jnp.maximum(v, 0.0)
                        return 0
                    lax.fori_loop(0, HV // UN, relu, 0)
                pltpu.async_copy(
                    buf.at[s], out_hbm.at[nz_idx.at[j]], ssems[s])
        return 0

    lax.fori_loop(0, lax.div(nzch + (NB - 1), jnp.int32(NB)), group, 0)

    def zfire(j, _):
        pltpu.async_copy(zbuf, out_hbm.at[z_idx.at[j]], zsem)
        return 0
    lax.fori_loop(jnp.minimum(2 * nzch, zch), zch, zfire, 0)

    for s in range(NB):
        j_s = (nzch - 1) - lax.rem(nzch - 1 - s + 6 * NZCH * NB, jnp.int32(NB))

        @pl.when(j_s >= 0)
        def _(j_s=j_s, s=s):
            pltpu.make_async_copy(
                buf.at[s], out_hbm.at[nz_idx.at[j_s]], ssems[s]).wait()

    def zdrain(j, _):
        pltpu.make_async_copy(zbuf, out_hbm.at[z_idx.at[j]], zsem).wait()
        return 0
    lax.fori_loop(0, zch, zdrain, 0)


@jax.jit
def _masked_relu(x, m):
    mesh = plsc.VectorSubcoreMesh(core_axis_name="c", subcore_axis_name="s")
    sc_out = pl.kernel(
        _sc_body,
        mesh=mesh,
        out_type=jax.ShapeDtypeStruct((NSC, H), jnp.float32),
        scratch_types=[
            pltpu.VMEM((RPW,), jnp.int32),
            pltpu.VMEM((NZCH, C), jnp.int32),
            pltpu.VMEM((NZCH, C), jnp.int32),
            pltpu.VMEM((ZCH, ZC), jnp.int32),
            pltpu.VMEM((NB, C, H), jnp.float32),
            pltpu.VMEM((ZC, H), jnp.float32),
            pltpu.SemaphoreType.DMA,
            pltpu.SemaphoreType.DMA,
            pltpu.SemaphoreType.DMA,
            pltpu.SemaphoreType.DMA,
            pltpu.SemaphoreType.DMA,
            pltpu.SemaphoreType.DMA,
            pltpu.SemaphoreType.DMA,
            pltpu.SemaphoreType.DMA,
            pltpu.SemaphoreType.DMA,
        ],
        compiler_params=pltpu.CompilerParams(needs_layout_passes=False),
    )(x, m)

    m2 = m.reshape(N, 1)
    tc_out = pl.pallas_call(
        _tc_body,
        grid=(S // BLK,),
        in_specs=[
            pl.BlockSpec((BLK, H), lambda i: (i, 0)),
            pl.BlockSpec((BLK, 1), lambda i: (i, 0)),
        ],
        out_specs=pl.BlockSpec((BLK, H), lambda i: (i, 0)),
        out_shape=jax.ShapeDtypeStruct((S, H), jnp.float32),
    )(x, m2)

    return jnp.concatenate([tc_out, sc_out], axis=0)


def kernel(inputs, mask):
    x = inputs.reshape(N, H)
    m = mask.reshape(N).astype(jnp.int32)
    out = _masked_relu(x, m)
    return out.reshape(inputs.shape)


# speedup vs baseline: 1.9864x; 1.9864x over previous
import jax
import jax.numpy as jnp
from jax import lax
from jax.experimental import pallas as pl
from jax.experimental.pallas import tpu as pltpu
from jax.experimental.pallas import tpu_sc as plsc

H = 2048
N = 32768
NC = 2
NS = 16
NW = NC * NS
RPW = N // NW
L = 16
C = 8
NB = 4
ZC = 8
NZCH = RPW // C
ZCH = RPW // ZC
HV = H // L
UN = 8


def _body(x_hbm, m_hbm, out_hbm, mask_v, nz_idx, z_idx, buf, zbuf,
          gs0, gs1, gs2, gs3, ss0, ss1, ss2, ss3, zsem):
    gsems = (gs0, gs1, gs2, gs3)
    ssems = (ss0, ss1, ss2, ss3)
    wid = lax.axis_index("s") * NC + lax.axis_index("c")
    base = wid * RPW

    pltpu.sync_copy(m_hbm.at[pl.ds(base, RPW)], mask_v)

    lanes1 = lax.iota(jnp.int32, L) + 1

    def compact(i, carry):
        nz_off, z_off = carry
        mv = mask_v[pl.ds(i * L, L)]
        rows = base + i * L + lax.iota(jnp.int32, L)
        pred = mv != 0
        predi = pred.astype(jnp.int32)
        cum = plsc.cumsum(predi)
        cnt = cum[L - 1]
        gpos = nz_off + cum - 1
        plsc.store_scatter(
            nz_idx,
            [jnp.right_shift(gpos, 3), jnp.bitwise_and(gpos, 7)],
            rows, mask=pred)
        zpred = jnp.logical_not(pred)
        zpos = z_off + (lanes1 - cum) - 1
        plsc.store_scatter(
            z_idx,
            [jnp.right_shift(zpos, 3), jnp.bitwise_and(zpos, 7)],
            rows, mask=zpred)
        return nz_off + cnt, z_off + (L - cnt)

    nz, z = lax.fori_loop(
        0, RPW // L, compact, (jnp.int32(0), jnp.int32(0)))


    lanes = lax.iota(jnp.int32, L)

    @pl.when((nz & (C - 1)) != 0)
    def _():
        zz = jnp.zeros((L,), jnp.int32)
        fill = plsc.load_gather(nz_idx, [zz, zz])
        row = jnp.zeros((L,), jnp.int32) + jnp.right_shift(nz, 3)
        plsc.store_scatter(
            nz_idx, [row, lanes], fill,
            mask=jnp.logical_and(lanes >= (nz & (C - 1)), lanes < C))

    @pl.when((z & (ZC - 1)) != 0)
    def _():
        zz = jnp.zeros((L,), jnp.int32)
        fill = plsc.load_gather(z_idx, [zz, zz])
        row = jnp.zeros((L,), jnp.int32) + jnp.right_shift(z, 3)
        plsc.store_scatter(
            z_idx, [row, lanes], fill,
            mask=jnp.logical_and(lanes >= (z & (ZC - 1)), lanes < ZC))

    nzch = jnp.right_shift(nz + (C - 1), 3)
    zch = jnp.right_shift(z + (ZC - 1), 3)

    zero = jnp.zeros((L,), jnp.float32)
    for r in range(ZC):
        def zinit(h, _, r=r):
            zbuf[r, pl.ds(h * L, L)] = zero
            return 0
        lax.fori_loop(0, HV, zinit, 0)

    @pl.when(nzch > 0)
    def _():
        pltpu.async_copy(x_hbm.at[nz_idx.at[0]], buf.at[0], gsems[0])

    def group(g, _):
        for s in range(NB):
            j = g * NB + s

            @pl.when(j < nzch)
            def _(j=j, s=s):
                s1 = (s + 1) % NB

                @pl.when(j + 1 < nzch)
                def _():
                    @pl.when(j + 1 >= NB)
                    def _():
                        pltpu.make_async_copy(
                            buf.at[s1], out_hbm.at[nz_idx.at[j + 1 - NB]],
                            ssems[s1]).wait()
                    pltpu.async_copy(
                        x_hbm.at[nz_idx.at[j + 1]], buf.at[s1], gsems[s1])

                for t in range(2):
                    @pl.when(2 * j + t < zch)
                    def _(t=t):
                        pltpu.async_copy(
                            zbuf, out_hbm.at[z_idx.at[2 * j + t]], zsem)

                pltpu.make_async_copy(
                    x_hbm.at[nz_idx.at[j]], buf.at[s], gsems[s]).wait()
                for r in range(C):
                    def relu(h, _, r=r, s=s):
                        for u in range(UN):
                            off = h * (L * UN) + u * L
                            v = buf[s, r, pl.ds(off, L)]
                            buf[s, r, pl.ds(off, L)] = jnp.maximum(v, 0.0)
                        return 0
                    lax.fori_loop(0, HV // UN, relu, 0)
                pltpu.async_copy(
                    buf.at[s], out_hbm.at[nz_idx.at[j]], ssems[s])
        return 0

    lax.fori_loop(0, lax.div(nzch + (NB - 1), jnp.int32(NB)), group, 0)

    def zfire(j, _):
        pltpu.async_copy(zbuf, out_hbm.at[z_idx.at[j]], zsem)
        return 0
    lax.fori_loop(jnp.minimum(2 * nzch, zch), zch, zfire, 0)

    for s in range(NB):
        j_s = (nzch - 1) - lax.rem(nzch - 1 - s + 6 * NZCH, jnp.int32(NB))

        @pl.when(j_s >= 0)
        def _(j_s=j_s, s=s):
            pltpu.make_async_copy(
                buf.at[s], out_hbm.at[nz_idx.at[j_s]], ssems[s]).wait()

    def zdrain(j, _):
        pltpu.make_async_copy(zbuf, out_hbm.at[z_idx.at[j]], zsem).wait()
        return 0
    lax.fori_loop(0, zch, zdrain, 0)


@jax.jit
def _masked_relu(x, m):
    mesh = plsc.VectorSubcoreMesh(core_axis_name="c", subcore_axis_name="s")
    return pl.kernel(
        _body,
        mesh=mesh,
        out_type=jax.ShapeDtypeStruct((N, H), jnp.float32),
        scratch_types=[
            pltpu.VMEM((RPW,), jnp.int32),
            pltpu.VMEM((NZCH, C), jnp.int32),
            pltpu.VMEM((ZCH, ZC), jnp.int32),
            pltpu.VMEM((NB, C, H), jnp.float32),
            pltpu.VMEM((ZC, H), jnp.float32),
            pltpu.SemaphoreType.DMA,
            pltpu.SemaphoreType.DMA,
            pltpu.SemaphoreType.DMA,
            pltpu.SemaphoreType.DMA,
            pltpu.SemaphoreType.DMA,
            pltpu.SemaphoreType.DMA,
            pltpu.SemaphoreType.DMA,
            pltpu.SemaphoreType.DMA,
            pltpu.SemaphoreType.DMA,
        ],
        compiler_params=pltpu.CompilerParams(needs_layout_passes=False),
    )(x, m)


def kernel(inputs, mask):
    x = inputs.reshape(N, H)
    m = mask.reshape(N).astype(jnp.int32)
    out = _masked_relu(x, m)
    return out.reshape(inputs.shape)


# speedup vs baseline: 1.9955x; 1.0046x over previous
import jax
import jax.numpy as jnp
from jax import lax
from jax.experimental import pallas as pl
from jax.experimental.pallas import tpu as pltpu
from jax.experimental.pallas import tpu_sc as plsc

H = 2048
N = 32768
NC = 2
NS = 16
NW = NC * NS
RPW = N // NW
L = 16
C = 8
NB = 4
ZC = 8
NZCH = RPW // C
ZCH = RPW // ZC
HV = H // L
UN = 8


def _body(x_hbm, m_hbm, out_hbm, mask_v, nz_idx, z_idx, buf, zbuf,
          gs0, gs1, gs2, gs3, ss0, ss1, ss2, ss3, zsem):
    gsems = (gs0, gs1, gs2, gs3)
    ssems = (ss0, ss1, ss2, ss3)
    wid = lax.axis_index("s") * NC + lax.axis_index("c")
    base = wid * RPW

    mcopy = pltpu.async_copy(m_hbm.at[pl.ds(base, RPW)], mask_v, zsem)

    zero = jnp.zeros((L,), jnp.float32)
    for r in range(ZC):
        def zinit(h, _, r=r):
            zbuf[r, pl.ds(h * L, L)] = zero
            return 0
        lax.fori_loop(0, HV, zinit, 0)
    mcopy.wait()

    lanes1 = lax.iota(jnp.int32, L) + 1

    def compact(i, carry):
        nz_off, z_off = carry
        mv = mask_v[pl.ds(i * L, L)]
        rows = base + i * L + lax.iota(jnp.int32, L)
        pred = mv != 0
        predi = pred.astype(jnp.int32)
        cum = plsc.cumsum(predi)
        cnt = cum[L - 1]
        gpos = nz_off + cum - 1
        plsc.store_scatter(
            nz_idx,
            [jnp.right_shift(gpos, 3), jnp.bitwise_and(gpos, 7)],
            rows, mask=pred)
        zpred = jnp.logical_not(pred)
        zpos = z_off + (lanes1 - cum) - 1
        plsc.store_scatter(
            z_idx,
            [jnp.right_shift(zpos, 3), jnp.bitwise_and(zpos, 7)],
            rows, mask=zpred)
        return nz_off + cnt, z_off + (L - cnt)

    nz, z = lax.fori_loop(
        0, RPW // L, compact, (jnp.int32(0), jnp.int32(0)))


    lanes = lax.iota(jnp.int32, L)

    @pl.when((nz & (C - 1)) != 0)
    def _():
        zz = jnp.zeros((L,), jnp.int32)
        fill = plsc.load_gather(nz_idx, [zz, zz])
        row = jnp.zeros((L,), jnp.int32) + jnp.right_shift(nz, 3)
        plsc.store_scatter(
            nz_idx, [row, lanes], fill,
            mask=jnp.logical_and(lanes >= (nz & (C - 1)), lanes < C))

    @pl.when((z & (ZC - 1)) != 0)
    def _():
        zz = jnp.zeros((L,), jnp.int32)
        fill = plsc.load_gather(z_idx, [zz, zz])
        row = jnp.zeros((L,), jnp.int32) + jnp.right_shift(z, 3)
        plsc.store_scatter(
            z_idx, [row, lanes], fill,
            mask=jnp.logical_and(lanes >= (z & (ZC - 1)), lanes < ZC))

    nzch = jnp.right_shift(nz + (C - 1), 3)
    zch = jnp.right_shift(z + (ZC - 1), 3)

    @pl.when(nzch > 0)
    def _():
        pltpu.async_copy(x_hbm.at[nz_idx.at[0]], buf.at[0], gsems[0])

    def group(g, _):
        for s in range(NB):
            j = g * NB + s

            @pl.when(j < nzch)
            def _(j=j, s=s):
                s1 = (s + 1) % NB

                @pl.when(j + 1 < nzch)
                def _():
                    @pl.when(j + 1 >= NB)
                    def _():
                        pltpu.make_async_copy(
                            buf.at[s1], out_hbm.at[nz_idx.at[j + 1 - NB]],
                            ssems[s1]).wait()
                    pltpu.async_copy(
                        x_hbm.at[nz_idx.at[j + 1]], buf.at[s1], gsems[s1])

                for t in range(2):
                    @pl.when(2 * j + t < zch)
                    def _(t=t):
                        pltpu.async_copy(
                            zbuf, out_hbm.at[z_idx.at[2 * j + t]], zsem)

                pltpu.make_async_copy(
                    x_hbm.at[nz_idx.at[j]], buf.at[s], gsems[s]).wait()
                for r in range(C):
                    def relu(h, _, r=r, s=s):
                        for u in range(UN):
                            off = h * (L * UN) + u * L
                            v = buf[s, r, pl.ds(off, L)]
                            buf[s, r, pl.ds(off, L)] = jnp.maximum(v, 0.0)
                        return 0
                    lax.fori_loop(0, HV // UN, relu, 0)
                pltpu.async_copy(
                    buf.at[s], out_hbm.at[nz_idx.at[j]], ssems[s])
        return 0

    lax.fori_loop(0, lax.div(nzch + (NB - 1), jnp.int32(NB)), group, 0)

    def zfire(j, _):
        pltpu.async_copy(zbuf, out_hbm.at[z_idx.at[j]], zsem)
        return 0
    lax.fori_loop(jnp.minimum(2 * nzch, zch), zch, zfire, 0)

    for s in range(NB):
        j_s = (nzch - 1) - lax.rem(nzch - 1 - s + 6 * NZCH, jnp.int32(NB))

        @pl.when(j_s >= 0)
        def _(j_s=j_s, s=s):
            pltpu.make_async_copy(
                buf.at[s], out_hbm.at[nz_idx.at[j_s]], ssems[s]).wait()

    def zdrain(j, _):
        pltpu.make_async_copy(zbuf, out_hbm.at[z_idx.at[j]], zsem).wait()
        return 0
    lax.fori_loop(0, zch, zdrain, 0)


@jax.jit
def _masked_relu(x, m):
    mesh = plsc.VectorSubcoreMesh(core_axis_name="c", subcore_axis_name="s")
    return pl.kernel(
        _body,
        mesh=mesh,
        out_type=jax.ShapeDtypeStruct((N, H), jnp.float32),
        scratch_types=[
            pltpu.VMEM((RPW,), jnp.int32),
            pltpu.VMEM((NZCH, C), jnp.int32),
            pltpu.VMEM((ZCH, ZC), jnp.int32),
            pltpu.VMEM((NB, C, H), jnp.float32),
            pltpu.VMEM((ZC, H), jnp.float32),
            pltpu.SemaphoreType.DMA,
            pltpu.SemaphoreType.DMA,
            pltpu.SemaphoreType.DMA,
            pltpu.SemaphoreType.DMA,
            pltpu.SemaphoreType.DMA,
            pltpu.SemaphoreType.DMA,
            pltpu.SemaphoreType.DMA,
            pltpu.SemaphoreType.DMA,
            pltpu.SemaphoreType.DMA,
        ],
        compiler_params=pltpu.CompilerParams(needs_layout_passes=False),
    )(x, m)


def kernel(inputs, mask):
    x = inputs.reshape(N, H)
    m = mask.reshape(N).astype(jnp.int32)
    out = _masked_relu(x, m)
    return out.reshape(inputs.shape)
